# baseline (device time: 101393 ns/iter reference)
import jax
import jax.numpy as jnp
from jax import lax
from jax.experimental import pallas as pl
from jax.experimental.pallas import tpu as pltpu

N_DEV = 8
FWD = 4
BWD = 3
B_LOC = 2
H_LOC = 4
SQ = 128
DH = 64
D_MODEL = 512
D_CHUNK = H_LOC * DH

_SLOT_OFFSETS = {0: 0, 1: -1, 2: -2, 3: -3, 4: -4, 5: 1, 6: 2, 7: 3}
_ISSUE_ORDER = (0, 1, 5, 2, 6, 3, 7, 4)


def kernel(x, Wq, K_ext, V_ext, Wo):
    x_b = x.astype(jnp.bfloat16)
    chunk = jnp.concatenate(
        [Wq.astype(jnp.bfloat16), Wo.T.astype(jnp.bfloat16)], axis=0)

    def body(x_ref, chunk_ref, k_hbm, v_hbm, out_ref,
             comm, kv, send_f, recv_f, send_b, recv_b, kv_sems):
        my_pos = lax.axis_index("i")
        left = jnp.mod(my_pos - 1, N_DEV)
        right = jnp.mod(my_pos + 1, N_DEV)

        kv_dmas = {}
        for s in _ISSUE_ORDER:
            origin = jnp.mod(my_pos + _SLOT_OFFSETS[s], N_DEV)
            dmas = []
            for t, hbm in ((0, k_hbm), (1, v_hbm)):
                for b in range(B_LOC):
                    b_glob = my_pos * B_LOC + b
                    for h in range(H_LOC):
                        dma = pltpu.make_async_copy(
                            hbm.at[b_glob, :, origin * H_LOC + h, :],
                            kv.at[t, s, b, h],
                            kv_sems.at[s],
                        )
                        dma.start()
                        dmas.append(dma)
            kv_dmas[s] = dmas

        barrier_sem = pltpu.get_barrier_semaphore()
        for nbr in (left, right):
            pl.semaphore_signal(
                barrier_sem, inc=1,
                device_id=(nbr,), device_id_type=pl.DeviceIdType.MESH,
            )
        pl.semaphore_wait(barrier_sem, 2)

        comm[0] = chunk_ref[...]

        qb = lax.broadcasted_iota(jnp.int32, (SQ, SQ), 0) // 64
        kb = lax.broadcasted_iota(jnp.int32, (SQ, SQ), 1) // 64
        mask = (qb == kb) | ((kb % 4) == (qb % 4))

        def compute_chunk(slot, first=False):
            for d in kv_dmas.pop(slot):
                d.wait()
            wq_c = comm[slot, :D_MODEL, :]
            woT_c = comm[slot, D_MODEL:, :]
            for b in range(B_LOC):
                q_full = jnp.dot(x_ref[b], wq_c,
                                 preferred_element_type=jnp.float32)
                ctx_parts = []
                for h in range(H_LOC):
                    q = q_full[:, h * DH:(h + 1) * DH]
                    k = kv[0, slot, b, h]
                    v = kv[1, slot, b, h]
                    sc = lax.dot_general(
                        q, k, (((1,), (1,)), ((), ())),
                        preferred_element_type=jnp.float32) * 0.125
                    sc = jnp.where(mask, sc, -1e9)
                    m = jnp.max(sc, axis=-1, keepdims=True)
                    w = jnp.exp(sc - m)
                    w = w / jnp.sum(w, axis=-1, keepdims=True)
                    ctx_parts.append(
                        jnp.dot(w, v, preferred_element_type=jnp.float32))
                ctx = jnp.concatenate(ctx_parts, axis=-1).astype(jnp.bfloat16)
                contrib = lax.dot_general(
                    ctx, woT_c, (((1,), (1,)), ((), ())),
                    preferred_element_type=jnp.float32)
                if first:
                    out_ref[b] = contrib
                else:
                    out_ref[b] = out_ref[b] + contrib

        for r in range(1, FWD + 1):
            rf = pltpu.make_async_remote_copy(
                src_ref=comm.at[r - 1], dst_ref=comm.at[r],
                send_sem=send_f.at[r - 1], recv_sem=recv_f.at[r - 1],
                device_id=(right,), device_id_type=pl.DeviceIdType.MESH,
            )
            rf.start()
            if r <= BWD:
                rb = pltpu.make_async_remote_copy(
                    src_ref=comm.at[0 if r == 1 else 4 + (r - 1)],
                    dst_ref=comm.at[4 + r],
                    send_sem=send_b.at[r - 1], recv_sem=recv_b.at[r - 1],
                    device_id=(left,), device_id_type=pl.DeviceIdType.MESH,
                )
                rb.start()
            if r == 1:
                compute_chunk(0, first=True)
            else:
                compute_chunk(r - 1)
                compute_chunk(4 + (r - 1))
            rf.wait()
            if r <= BWD:
                rb.wait()
        compute_chunk(FWD)

    out_shape = jax.ShapeDtypeStruct((B_LOC, SQ, D_MODEL), jnp.float32)
    return pl.pallas_call(
        body,
        out_shape=out_shape,
        in_specs=[
            pl.BlockSpec(memory_space=pltpu.VMEM),
            pl.BlockSpec(memory_space=pltpu.VMEM),
            pl.BlockSpec(memory_space=pltpu.MemorySpace.HBM),
            pl.BlockSpec(memory_space=pltpu.MemorySpace.HBM),
        ],
        out_specs=pl.BlockSpec(memory_space=pltpu.VMEM),
        scratch_shapes=[
            pltpu.VMEM((N_DEV, 2 * D_MODEL, D_CHUNK), jnp.bfloat16),
            pltpu.VMEM((2, N_DEV, B_LOC, H_LOC, SQ, DH), jnp.float32),
            pltpu.SemaphoreType.DMA((FWD,)),
            pltpu.SemaphoreType.DMA((FWD,)),
            pltpu.SemaphoreType.DMA((BWD,)),
            pltpu.SemaphoreType.DMA((BWD,)),
            pltpu.SemaphoreType.DMA((N_DEV,)),
        ],
        compiler_params=pltpu.CompilerParams(collective_id=0),
    )(x_b, chunk, K_ext, V_ext)


# device time: 46196 ns/iter; 2.1948x vs baseline; 2.1948x over previous
import jax
import jax.numpy as jnp
from jax import lax
from jax.experimental import pallas as pl
from jax.experimental.pallas import tpu as pltpu

N_DEV = 8
FWD = 4
BWD = 3
B_LOC = 2
H_LOC = 4
SQ = 128
SKV = 128
HQ = 32
DH = 64
D_MODEL = 512
D_CHUNK = H_LOC * DH


def kernel(x, Wq, K_ext, V_ext, Wo):
    my = lax.axis_index("i")
    k2 = K_ext.reshape(K_ext.shape[0], SKV, HQ * DH)
    v2 = V_ext.reshape(V_ext.shape[0], SKV, HQ * DH)
    k_loc = lax.dynamic_slice_in_dim(k2, my * B_LOC, B_LOC, axis=0)
    v_loc = lax.dynamic_slice_in_dim(v2, my * B_LOC, B_LOC, axis=0)
    k_b = k_loc.astype(jnp.bfloat16)
    v_b = v_loc.astype(jnp.bfloat16)
    x_b = x.astype(jnp.bfloat16)
    chunk = jnp.concatenate(
        [Wq.astype(jnp.bfloat16), Wo.T.astype(jnp.bfloat16)], axis=0)

    def body(x_ref, chunk_ref, k_ref, v_ref, out_ref,
             comm, send_f, recv_f, send_b, recv_b):
        my_pos = lax.axis_index("i")
        left = jnp.mod(my_pos - 1, N_DEV)
        right = jnp.mod(my_pos + 1, N_DEV)

        barrier_sem = pltpu.get_barrier_semaphore()
        for nbr in (left, right):
            pl.semaphore_signal(
                barrier_sem, inc=1,
                device_id=(nbr,), device_id_type=pl.DeviceIdType.MESH,
            )
        pl.semaphore_wait(barrier_sem, 2)

        comm[0] = chunk_ref[...]

        qb = lax.broadcasted_iota(jnp.int32, (SQ, SQ), 0) // 64
        kb = lax.broadcasted_iota(jnp.int32, (SQ, SQ), 1) // 64
        mask = (qb == kb) | ((kb % 4) == (qb % 4))

        def compute_chunk(slot, origin, first=False):
            src = jnp.mod(origin, N_DEV)
            lane0 = src * D_CHUNK
            wq_c = comm[slot, :D_MODEL, :]
            woT_c = comm[slot, D_MODEL:, :]
            for b in range(B_LOC):
                q_full = (jnp.dot(x_ref[b], wq_c,
                                  preferred_element_type=jnp.float32)
                          ).astype(jnp.bfloat16)
                k4 = k_ref[b, :, pl.ds(lane0, D_CHUNK)]
                v4 = v_ref[b, :, pl.ds(lane0, D_CHUNK)]
                ctx_parts = []
                for h in range(H_LOC):
                    sl = slice(h * DH, (h + 1) * DH)
                    sc = lax.dot_general(
                        q_full[:, sl], k4[:, sl], (((1,), (1,)), ((), ())),
                        preferred_element_type=jnp.float32) * 0.125
                    sc = jnp.where(mask, sc, -1e9)
                    m = jnp.max(sc, axis=-1, keepdims=True)
                    w = jnp.exp(sc - m)
                    w = (w / jnp.sum(w, axis=-1, keepdims=True)
                         ).astype(jnp.bfloat16)
                    ctx_parts.append(
                        jnp.dot(w, v4[:, sl],
                                preferred_element_type=jnp.float32))
                ctx = jnp.concatenate(ctx_parts, axis=-1).astype(jnp.bfloat16)
                contrib = lax.dot_general(
                    ctx, woT_c, (((1,), (1,)), ((), ())),
                    preferred_element_type=jnp.float32)
                if first:
                    out_ref[b] = contrib
                else:
                    out_ref[b] = out_ref[b] + contrib

        for r in range(1, FWD + 1):
            rf = pltpu.make_async_remote_copy(
                src_ref=comm.at[r - 1], dst_ref=comm.at[r],
                send_sem=send_f.at[r - 1], recv_sem=recv_f.at[r - 1],
                device_id=(right,), device_id_type=pl.DeviceIdType.MESH,
            )
            rf.start()
            if r <= BWD:
                rb = pltpu.make_async_remote_copy(
                    src_ref=comm.at[0 if r == 1 else 4 + (r - 1)],
                    dst_ref=comm.at[4 + r],
                    send_sem=send_b.at[r - 1], recv_sem=recv_b.at[r - 1],
                    device_id=(left,), device_id_type=pl.DeviceIdType.MESH,
                )
                rb.start()
            if r == 1:
                compute_chunk(0, my_pos, first=True)
            else:
                compute_chunk(r - 1, my_pos - (r - 1))
                compute_chunk(4 + (r - 1), my_pos + (r - 1))
            rf.wait()
            if r <= BWD:
                rb.wait()
        compute_chunk(FWD, my_pos - FWD)

    out_shape = jax.ShapeDtypeStruct((B_LOC, SQ, D_MODEL), jnp.float32)
    return pl.pallas_call(
        body,
        out_shape=out_shape,
        in_specs=[pl.BlockSpec(memory_space=pltpu.VMEM)] * 4,
        out_specs=pl.BlockSpec(memory_space=pltpu.VMEM),
        scratch_shapes=[
            pltpu.VMEM((N_DEV, 2 * D_MODEL, D_CHUNK), jnp.bfloat16),
            pltpu.SemaphoreType.DMA((FWD,)),
            pltpu.SemaphoreType.DMA((FWD,)),
            pltpu.SemaphoreType.DMA((BWD,)),
            pltpu.SemaphoreType.DMA((BWD,)),
        ],
        compiler_params=pltpu.CompilerParams(collective_id=0),
    )(x_b, chunk, k_b, v_b)
